# Initial kernel scaffold; baseline (speedup 1.0000x reference)
import jax
import jax.numpy as jnp
from jax.experimental import pallas as pl


def kernel(value, value_spatial_shapes, sampling_locations, attention_weights):
    raise NotImplementedError("write your pallas kernel here")


# speedup vs baseline: 1008.1961x; 1008.1961x over previous
import functools

import jax
import jax.numpy as jnp
from jax import lax
from jax.experimental import pallas as pl
from jax.experimental.pallas import tpu as pltpu
from jax.experimental.pallas import tpu_sc as plsc

H, W = 50, 50
LANES = 16
QB = 512


def _c(v):
  return jnp.full((LANES,), v, jnp.int32)


def _deform_body(nq, nh, ed, npts, bh_per_worker, nblocks, num_cores,
                 val_ref, loc_ref, att_ref, out_ref,
                 table_v, loc_v, att_v, out_v):
  wid = lax.axis_index("s") * num_cores + lax.axis_index("c")
  lanes = lax.iota(jnp.int32, LANES)

  @pl.loop(0, bh_per_worker)
  def _bh_loop(t):
    bh = wid * bh_per_worker + t
    b = bh // nh
    h = bh % nh
    pltpu.sync_copy(val_ref.at[b, :, h, :], table_v)

    @pl.loop(0, nblocks)
    def _block_loop(blk):
      q0 = jnp.minimum(blk * QB, nq - QB)
      pltpu.sync_copy(loc_ref.at[b, pl.ds(q0, QB), h], loc_v)
      pltpu.sync_copy(att_ref.at[b, pl.ds(q0, QB), h], att_v)

      @pl.loop(0, QB // LANES)
      def _chunk_loop(ck):
        qv = ck * LANES + lanes
        acc = [jnp.zeros((LANES,), jnp.float32)] * ed
        for p in range(npts):
          lx = plsc.load_gather(loc_v, [qv, _c(2 * p)])
          ly = plsc.load_gather(loc_v, [qv, _c(2 * p + 1)])
          aw = plsc.load_gather(att_v, [qv, _c(p)])
          x = lx * float(W) - 0.5
          y = ly * float(H) - 0.5
          x0 = (x + 1.0).astype(jnp.int32) - 1
          y0 = (y + 1.0).astype(jnp.int32) - 1
          fx = x - x0.astype(jnp.float32)
          fy = y - y0.astype(jnp.float32)
          x1 = x0 + 1
          y1 = y0 + 1
          wx = (1.0 - fx, fx)
          wy = (1.0 - fy, fy)
          vx = (x0 >= 0, x1 <= W - 1)
          vy = (y0 >= 0, y1 <= H - 1)
          cx = (jnp.clip(x0, 0, W - 1), jnp.clip(x1, 0, W - 1))
          cy = (jnp.clip(y0, 0, H - 1), jnp.clip(y1, 0, H - 1))
          for iy in range(2):
            for ix in range(2):
              wgt = jnp.where(vx[ix] & vy[iy], wx[ix] * wy[iy] * aw, 0.0)
              row = cy[iy] * W + cx[ix]
              for f in range(ed):
                acc[f] = acc[f] + wgt * plsc.load_gather(table_v, [row, _c(f)])
        for f in range(ed):
          plsc.store_scatter(out_v, [qv, _c(f)], acc[f])

      pltpu.sync_copy(out_v, out_ref.at[b, pl.ds(q0, QB), h])


def kernel(value, value_spatial_shapes, sampling_locations, attention_weights):
  del value_spatial_shapes
  bs, nk, nh, ed = value.shape
  nq = sampling_locations.shape[1]
  npts = sampling_locations.shape[4]
  loc = sampling_locations.reshape(bs, nq, nh, npts * 2)
  att = attention_weights.reshape(bs, nq, nh, npts)

  info = plsc.get_sparse_core_info()
  num_cores, num_subcores = info.num_cores, info.num_subcores
  nworkers = num_cores * num_subcores
  assert (bs * nh) % nworkers == 0
  bh_per_worker = (bs * nh) // nworkers
  nblocks = (nq + QB - 1) // QB

  mesh = plsc.VectorSubcoreMesh(core_axis_name="c", subcore_axis_name="s")
  body = functools.partial(_deform_body, nq, nh, ed, npts, bh_per_worker,
                           nblocks, num_cores)
  out = pl.kernel(
      body,
      out_type=jax.ShapeDtypeStruct((bs, nq, nh, ed), jnp.float32),
      mesh=mesh,
      compiler_params=pltpu.CompilerParams(
          needs_layout_passes=False, use_tc_tiling_on_sc=False),
      scratch_types=[
          pltpu.VMEM((nk, ed), jnp.float32),
          pltpu.VMEM((QB, npts * 2), jnp.float32),
          pltpu.VMEM((QB, npts), jnp.float32),
          pltpu.VMEM((QB, ed), jnp.float32),
      ],
  )(value, loc, att)
  return out.reshape(bs, nq, nh * ed)


# speedup vs baseline: 1011.3479x; 1.0031x over previous
import functools

import jax
import jax.numpy as jnp
from jax import lax
from jax.experimental import pallas as pl
from jax.experimental.pallas import tpu as pltpu
from jax.experimental.pallas import tpu_sc as plsc

H, W = 50, 50
LANES = 16
QB = 512


def _c(v):
  return jnp.full((LANES,), v, jnp.int32)


def _deform_body(nq, nh, ed, npts, bh_per_worker, nblocks, num_cores,
                 val_ref, loc_ref, att_ref, out_ref,
                 table_v, loc_v, att_v, out_v):
  wid = lax.axis_index("s") * num_cores + lax.axis_index("c")
  lanes = lax.iota(jnp.int32, LANES)

  @pl.loop(0, bh_per_worker)
  def _bh_loop(t):
    bh = wid * bh_per_worker + t
    b = bh // nh
    h = bh % nh
    with jax.named_scope("table_dma"):
      pltpu.sync_copy(val_ref.at[b, :, h, :], table_v)

    @pl.loop(0, nblocks)
    def _block_loop(blk):
      q0 = jnp.minimum(blk * QB, nq - QB)
      with jax.named_scope("in_dma"):
        pltpu.sync_copy(loc_ref.at[b, pl.ds(q0, QB), h], loc_v)
        pltpu.sync_copy(att_ref.at[b, pl.ds(q0, QB), h], att_v)

      @pl.loop(0, QB // LANES)
      def _chunk_loop(ck):
        qv = ck * LANES + lanes
        acc = [jnp.zeros((LANES,), jnp.float32)] * ed
        for p in range(npts):
          lx = plsc.load_gather(loc_v, [qv, _c(2 * p)])
          ly = plsc.load_gather(loc_v, [qv, _c(2 * p + 1)])
          aw = plsc.load_gather(att_v, [qv, _c(p)])
          x = lx * float(W) - 0.5
          y = ly * float(H) - 0.5
          x0 = (x + 1.0).astype(jnp.int32) - 1
          y0 = (y + 1.0).astype(jnp.int32) - 1
          fx = x - x0.astype(jnp.float32)
          fy = y - y0.astype(jnp.float32)
          x1 = x0 + 1
          y1 = y0 + 1
          wx = (1.0 - fx, fx)
          wy = (1.0 - fy, fy)
          vx = (x0 >= 0, x1 <= W - 1)
          vy = (y0 >= 0, y1 <= H - 1)
          cx = (jnp.clip(x0, 0, W - 1), jnp.clip(x1, 0, W - 1))
          cy = (jnp.clip(y0, 0, H - 1), jnp.clip(y1, 0, H - 1))
          for iy in range(2):
            for ix in range(2):
              wgt = jnp.where(vx[ix] & vy[iy], wx[ix] * wy[iy] * aw, 0.0)
              row = cy[iy] * W + cx[ix]
              for f in range(ed):
                acc[f] = acc[f] + wgt * plsc.load_gather(table_v, [row, _c(f)])
        for f in range(ed):
          plsc.store_scatter(out_v, [qv, _c(f)], acc[f])

      with jax.named_scope("out_dma"):
        pltpu.sync_copy(out_v, out_ref.at[b, pl.ds(q0, QB), h])


def kernel(value, value_spatial_shapes, sampling_locations, attention_weights):
  del value_spatial_shapes
  bs, nk, nh, ed = value.shape
  nq = sampling_locations.shape[1]
  npts = sampling_locations.shape[4]
  loc = sampling_locations.reshape(bs, nq, nh, npts * 2)
  att = attention_weights.reshape(bs, nq, nh, npts)

  info = plsc.get_sparse_core_info()
  num_cores, num_subcores = info.num_cores, info.num_subcores
  nworkers = num_cores * num_subcores
  assert (bs * nh) % nworkers == 0
  bh_per_worker = (bs * nh) // nworkers
  nblocks = (nq + QB - 1) // QB

  mesh = plsc.VectorSubcoreMesh(core_axis_name="c", subcore_axis_name="s")
  body = functools.partial(_deform_body, nq, nh, ed, npts, bh_per_worker,
                           nblocks, num_cores)
  out = pl.kernel(
      body,
      out_type=jax.ShapeDtypeStruct((bs, nq, nh, ed), jnp.float32),
      mesh=mesh,
      compiler_params=pltpu.CompilerParams(
          needs_layout_passes=False, use_tc_tiling_on_sc=False),
      scratch_types=[
          pltpu.VMEM((nk, ed), jnp.float32),
          pltpu.VMEM((QB, npts * 2), jnp.float32),
          pltpu.VMEM((QB, npts), jnp.float32),
          pltpu.VMEM((QB, ed), jnp.float32),
      ],
  )(value, loc, att)
  return out.reshape(bs, nq, nh * ed)
